# Initial kernel scaffold; baseline (speedup 1.0000x reference)
import jax
import jax.numpy as jnp
from jax.experimental import pallas as pl


def kernel(encoded_words, embed_weight, pos_emb_weight):
    raise NotImplementedError("write your pallas kernel here")


# speedup vs baseline: 2.4283x; 2.4283x over previous
import functools

import jax
import jax.numpy as jnp
from jax import lax
from jax.experimental import pallas as pl
from jax.experimental.pallas import tpu as pltpu
from jax.experimental.pallas import tpu_sc as plsc

VOCAB = 1000000
D = 64
SEQ = 200
BATCH = 4096

NC = 2
NS = 16
NW = NC * NS

CHUNK = 128
TOT = BATCH * SEQ
NCHUNK = TOT // CHUNK
CPW = NCHUNK // NW
NBUF = 4
LANES = 16

_mesh = plsc.VectorSubcoreMesh(core_axis_name="c", subcore_axis_name="s")


@functools.partial(
    pl.kernel,
    mesh=_mesh,
    compiler_params=pltpu.CompilerParams(use_tc_tiling_on_sc=False),
    out_type=jax.ShapeDtypeStruct((NCHUNK, CHUNK, D), jnp.float32),
    scratch_types=[
        pltpu.VMEM((CPW, CHUNK), jnp.int32),
        pltpu.VMEM((SEQ, D), jnp.float32),
        pltpu.VMEM((NBUF, CHUNK, D), jnp.float32),
        pltpu.SemaphoreType.DMA((NBUF,)),
        pltpu.SemaphoreType.DMA((NBUF,)),
    ],
)
def _emb_kernel(table_hbm, idx_hbm, pos_hbm, out_hbm,
                idx_v, pos_v, rows_v, gsem, osem):
    wid = lax.axis_index("s") * NC + lax.axis_index("c")
    row0 = wid * CPW

    pltpu.make_async_copy(
        idx_hbm.at[pl.ds(row0, CPW)], idx_v, gsem.at[0]).start()
    pltpu.make_async_copy(
        pos_hbm.at[pl.ds(0, SEQ)], pos_v, osem.at[0]).start()
    pltpu.make_async_copy(
        idx_hbm.at[pl.ds(row0, CPW)], idx_v, gsem.at[0]).wait()
    pltpu.make_async_copy(
        pos_hbm.at[pl.ds(0, SEQ)], pos_v, osem.at[0]).wait()

    def start_gather(j, b):
        pltpu.make_async_copy(
            table_hbm.at[idx_v.at[j]], rows_v.at[b], gsem.at[b]).start()

    def wait_gather(j, b):
        pltpu.make_async_copy(
            table_hbm.at[idx_v.at[j]], rows_v.at[b], gsem.at[b]).wait()

    def start_out(j, b):
        pltpu.make_async_copy(
            rows_v.at[b], out_hbm.at[row0 + j], osem.at[b]).start()

    def wait_out(j, b):
        pltpu.make_async_copy(
            rows_v.at[b], out_hbm.at[row0 + j], osem.at[b]).wait()

    def add_pos(j, b):
        p0 = lax.rem(j * CHUNK, SEQ)

        def row_body(r, p):
            for c in range(D // LANES):
                sl = pl.ds(c * LANES, LANES)
                plsc.addupdate(rows_v.at[b, r, sl], pos_v[p, sl])
            p = p + 1
            return jnp.where(p == SEQ, 0, p)

        lax.fori_loop(0, CHUNK, row_body, p0, unroll=2)

    def step(j, b, first_round):
        wait_gather(j, b)
        add_pos(j, b)
        start_out(j, b)
        pb = (b - 1) % NBUF
        if first_round:
            if b != 0:
                wait_out(j - 1, pb)
            start_gather(j + NBUF - 1, pb)
        else:
            @pl.when(j + NBUF - 1 < CPW)
            def _():
                wait_out(j - 1, pb)
                start_gather(j + NBUF - 1, pb)

    for b in range(NBUF - 1):
        start_gather(b, b)

    for b in range(NBUF):
        step(b, b, first_round=True)

    def outer(g, carry):
        for b in range(NBUF):
            step(g * NBUF + b, b, first_round=False)
        return carry

    lax.fori_loop(1, CPW // NBUF, outer, 0)

    for b in range(NBUF):
        wait_out(CPW - NBUF + b, b)


def kernel(encoded_words, embed_weight, pos_emb_weight):
    idx = encoded_words.astype(jnp.int32).reshape(NCHUNK, CHUNK)
    out = _emb_kernel(embed_weight, idx, pos_emb_weight)
    return out.reshape(BATCH, SEQ, D)


# speedup vs baseline: 2.8608x; 1.1781x over previous
import functools

import jax
import jax.numpy as jnp
from jax import lax
from jax.experimental import pallas as pl
from jax.experimental.pallas import tpu as pltpu
from jax.experimental.pallas import tpu_sc as plsc

VOCAB = 1000000
D = 64
SEQ = 200
BATCH = 4096

NC = 2
NS = 16
NW = NC * NS

SPW = BATCH // NW
NBUF = 4
LANES = 16
H0 = 104
H1 = SEQ - H0

_mesh = plsc.VectorSubcoreMesh(core_axis_name="c", subcore_axis_name="s")


@functools.partial(
    pl.kernel,
    mesh=_mesh,
    compiler_params=pltpu.CompilerParams(use_tc_tiling_on_sc=False),
    out_type=jax.ShapeDtypeStruct((BATCH, SEQ, D), jnp.float32),
    scratch_types=[
        pltpu.VMEM((SPW, SEQ), jnp.int32),
        pltpu.VMEM((SEQ, D), jnp.float32),
        pltpu.VMEM((NBUF, SEQ, D), jnp.float32),
        pltpu.SemaphoreType.DMA((NBUF,)),
        pltpu.SemaphoreType.DMA((NBUF,)),
    ],
)
def _emb_kernel(table_hbm, idx_hbm, pos_hbm, out_hbm,
                idx_v, pos_v, rows_v, gsem, osem):
    wid = lax.axis_index("s") * NC + lax.axis_index("c")
    seq0 = wid * SPW

    pltpu.make_async_copy(
        idx_hbm.at[pl.ds(seq0, SPW)], idx_v, gsem.at[0]).start()
    pltpu.make_async_copy(
        pos_hbm.at[pl.ds(0, SEQ)], pos_v, osem.at[0]).start()
    pltpu.make_async_copy(
        idx_hbm.at[pl.ds(seq0, SPW)], idx_v, gsem.at[0]).wait()
    pltpu.make_async_copy(
        pos_hbm.at[pl.ds(0, SEQ)], pos_v, osem.at[0]).wait()

    def start_gather(j, b):
        pltpu.make_async_copy(
            table_hbm.at[idx_v.at[j, pl.ds(0, H0)]],
            rows_v.at[b, pl.ds(0, H0)], gsem.at[b]).start()
        pltpu.make_async_copy(
            table_hbm.at[idx_v.at[j, pl.ds(H0, H1)]],
            rows_v.at[b, pl.ds(H0, H1)], gsem.at[b]).start()

    def wait_gather(j, b):
        pltpu.make_async_copy(
            table_hbm.at[idx_v.at[j, pl.ds(0, H0)]],
            rows_v.at[b, pl.ds(0, H0)], gsem.at[b]).wait()
        pltpu.make_async_copy(
            table_hbm.at[idx_v.at[j, pl.ds(H0, H1)]],
            rows_v.at[b, pl.ds(H0, H1)], gsem.at[b]).wait()

    def start_out(j, b):
        pltpu.make_async_copy(
            rows_v.at[b], out_hbm.at[seq0 + j], osem.at[b]).start()

    def wait_out(j, b):
        pltpu.make_async_copy(
            rows_v.at[b], out_hbm.at[seq0 + j], osem.at[b]).wait()

    def add_pos(b):
        def row_body(r, carry):
            for c in range(D // LANES):
                sl = pl.ds(c * LANES, LANES)
                plsc.addupdate(rows_v.at[b, r, sl], pos_v[r, sl])
            return carry

        lax.fori_loop(0, SEQ, row_body, 0, unroll=4)

    def step(j, b, first_round):
        wait_gather(j, b)
        add_pos(b)
        start_out(j, b)
        pb = (b - 1) % NBUF
        if first_round:
            if b != 0:
                wait_out(j - 1, pb)
            start_gather(j + NBUF - 1, pb)
        else:
            @pl.when(j + NBUF - 1 < SPW)
            def _():
                wait_out(j - 1, pb)
                start_gather(j + NBUF - 1, pb)

    for b in range(NBUF - 1):
        start_gather(b, b)

    for b in range(NBUF):
        step(b, b, first_round=True)

    def outer(g, carry):
        for b in range(NBUF):
            step(g * NBUF + b, b, first_round=False)
        return carry

    lax.fori_loop(1, SPW // NBUF, outer, 0)

    for b in range(NBUF):
        wait_out(SPW - NBUF + b, b)


def kernel(encoded_words, embed_weight, pos_emb_weight):
    return _emb_kernel(embed_weight,
                       encoded_words.astype(jnp.int32),
                       pos_emb_weight)
